# baseline (speedup 1.0000x reference)
import functools

import jax
import jax.numpy as jnp
from jax import lax
from jax.experimental import pallas as pl
from jax.experimental.pallas import tpu as pltpu
from jax.experimental.pallas import tpu_sc as plsc

B = 16384
ND = 26
VOCAB = 100000
DE = 16
NWIDE = 26
WAD = 128
HALF = WAD // 2
E = 8
K = 2
CAP = 4096

NWORK = 32
NCHUNK = 4
PF = 32
TOTAL_ROWS = B * PF
RPC = TOTAL_ROWS // (NWORK * NCHUNK)

FD = ND * DE
NG = 4
CB = 12544
NCB = 8
VP = NCB * CB


def _repack_body(t_ref, p_ref):
    p_ref[...] = t_ref[...].T


def _repack(tabT):
    return pl.pallas_call(
        _repack_body,
        grid=(NG, NCB),
        in_specs=[pl.BlockSpec((128, CB), lambda g, c: (g, c))],
        out_specs=pl.BlockSpec((CB, 128), lambda g, c: (g * NCB + c, 0)),
        out_shape=jax.ShapeDtypeStruct((NG * VP, 128), jnp.float32),
    )(tabT)

@functools.cache
def _make_sc_gather():
    mesh = plsc.VectorSubcoreMesh(
        core_axis_name="c", subcore_axis_name="s", num_cores=2, num_subcores=16
    )

    @functools.partial(
        pl.kernel,
        out_type=jax.ShapeDtypeStruct((NWORK * NCHUNK, RPC, DE), jnp.float32),
        mesh=mesh,
        scratch_types=[
            pltpu.VMEM((RPC,), jnp.int32),
            pltpu.VMEM((RPC, DE), jnp.float32),
            pltpu.SemaphoreType.DMA,
        ],
        compiler_params=pltpu.CompilerParams(use_tc_tiling_on_sc=False),
    )
    def sc_gather(tab_hbm, idx_hbm, out_hbm, idx_v, rows_v, sem):
        w = lax.axis_index("s") * 2 + lax.axis_index("c")
        for c in range(NCHUNK):
            k = w * NCHUNK + c
            pltpu.sync_copy(idx_hbm.at[k], idx_v)
            pltpu.async_copy(tab_hbm.at[idx_v], rows_v, sem).wait()
            pltpu.sync_copy(rows_v, out_hbm.at[k])

    return sc_gather


def _sc_gather(tab_flat, idx):
    return _make_sc_gather()(tab_flat, idx)


TB = 512
NB = B // TB


def _lrelu(v):
    return jnp.where(v >= 0, v, 0.2 * v)


def _ctx_router_body(deep_ref, wide_ref, Wd_ref, bd_ref, Ww_ref, bw_ref, Wr_ref,
                     x_ref, tv_ref, ti_ref, h0_ref, h1_ref, ps_ref):
    dc = jnp.concatenate([deep_ref[g] for g in range(NG)], axis=1)
    d = jnp.dot(dc, Wd_ref[...], preferred_element_type=jnp.float32)
    d = _lrelu(d + bd_ref[...])
    w = jnp.dot(wide_ref[...], Ww_ref[...], preferred_element_type=jnp.float32)
    w = _lrelu(w + bw_ref[...])
    x = jnp.concatenate([d, w], axis=1)
    x_ref[...] = x
    lg = jnp.dot(x, Wr_ref[...], preferred_element_type=jnp.float32)
    m = jnp.max(lg, axis=1, keepdims=True)
    ex = jnp.exp(lg - m)
    p = ex / jnp.sum(ex, axis=1, keepdims=True)
    ii = lax.broadcasted_iota(jnp.int32, p.shape, 1)
    v1 = jnp.max(p, axis=1, keepdims=True)
    i1 = jnp.min(jnp.where(p == v1, ii, E), axis=1, keepdims=True)
    p2 = jnp.where(ii == i1, -1.0, p)
    v2 = jnp.max(p2, axis=1, keepdims=True)
    i2 = jnp.min(jnp.where(p2 == v2, ii, E), axis=1, keepdims=True)
    tv_ref[...] = jnp.concatenate([v1, v2], axis=1)
    ti_ref[...] = jnp.concatenate([i1, i2], axis=1)
    oh0 = (ii == i1).astype(jnp.float32)
    oh1 = (ii == i2).astype(jnp.float32)
    h0_ref[...] = jnp.sum(oh0, axis=0).reshape(1, 1, E)
    h1_ref[...] = jnp.sum(oh1, axis=0).reshape(1, 1, E)
    ps_ref[...] = jnp.sum(p, axis=0).reshape(1, 1, E)


def _ctx_router(deep_cat, wide_T, Wd, bd, Ww, bw, Wr):
    return pl.pallas_call(
        _ctx_router_body,
        grid=(NB,),
        in_specs=[
            pl.BlockSpec((NG, TB, 8 * DE), lambda i: (0, i, 0)),
            pl.BlockSpec((TB, NWIDE), lambda i: (i, 0)),
            pl.BlockSpec((PF * DE, HALF), lambda i: (0, 0)),
            pl.BlockSpec((1, HALF), lambda i: (0, 0)),
            pl.BlockSpec((NWIDE, HALF), lambda i: (0, 0)),
            pl.BlockSpec((1, HALF), lambda i: (0, 0)),
            pl.BlockSpec((WAD, E), lambda i: (0, 0)),
        ],
        out_specs=[
            pl.BlockSpec((TB, WAD), lambda i: (i, 0)),
            pl.BlockSpec((TB, K), lambda i: (i, 0)),
            pl.BlockSpec((TB, K), lambda i: (i, 0)),
            pl.BlockSpec((1, 1, E), lambda i: (i, 0, 0)),
            pl.BlockSpec((1, 1, E), lambda i: (i, 0, 0)),
            pl.BlockSpec((1, 1, E), lambda i: (i, 0, 0)),
        ],
        out_shape=[
            jax.ShapeDtypeStruct((B, WAD), jnp.float32),
            jax.ShapeDtypeStruct((B, K), jnp.float32),
            jax.ShapeDtypeStruct((B, K), jnp.int32),
            jax.ShapeDtypeStruct((NB, 1, E), jnp.float32),
            jax.ShapeDtypeStruct((NB, 1, E), jnp.float32),
            jax.ShapeDtypeStruct((NB, 1, E), jnp.float32),
        ],
    )(deep_cat, wide_T, Wd, bd, Ww, bw, Wr)


def _moe_head_body(x_ref, tv_ref, ti_ref, h0_ref, h1_ref, ps_ref,
                   W1_ref, b1_ref, W2_ref, b2_ref, W0_ref, b0_ref,
                   W1d_ref, b1d_ref, WoT_ref, bo_ref, out_ref, aux_ref):
    i = pl.program_id(0)
    x = x_ref[...]
    tv = tv_ref[...]
    ti = ti_ref[...]
    ii = lax.broadcasted_iota(jnp.int32, (TB, E), 1)
    oh0 = (ii == ti[:, 0:1]).astype(jnp.float32)
    oh1 = (ii == ti[:, 1:2]).astype(jnp.float32)
    h0 = h0_ref[...]
    h1 = h1_ref[...]
    bmask = (lax.broadcasted_iota(jnp.int32, (NB, 1, E), 0) < i).astype(jnp.float32)
    prefix0 = jnp.sum(h0 * bmask, axis=(0, 1))
    prefix1 = jnp.sum(h1 * bmask, axis=(0, 1))
    total0 = jnp.sum(h0, axis=(0, 1))
    total1 = jnp.sum(h1, axis=(0, 1))
    r = lax.broadcasted_iota(jnp.int32, (TB, TB), 0)
    cc = lax.broadcasted_iota(jnp.int32, (TB, TB), 1)
    L = (cc < r).astype(jnp.float32)
    prior0 = lax.dot(L, oh0, precision=lax.Precision.HIGHEST,
                     preferred_element_type=jnp.float32)
    prior1 = lax.dot(L, oh1, precision=lax.Precision.HIGHEST,
                     preferred_element_type=jnp.float32)
    keep0 = oh0 * (prefix0[None, :] + prior0 < CAP).astype(jnp.float32)
    keep1 = oh1 * (total0[None, :] + prefix1[None, :] + prior1 < CAP).astype(jnp.float32)
    combine = tv[:, 0:1] * keep0 + tv[:, 1:2] * keep1
    moe = jnp.zeros((TB, WAD), jnp.float32)
    for e in range(E):
        h = jnp.dot(x, W1_ref[e], preferred_element_type=jnp.float32)
        h = jnp.maximum(h + b1_ref[e:e + 1, :], 0.0)
        y = jnp.dot(h, W2_ref[e], preferred_element_type=jnp.float32)
        y = y + b2_ref[e:e + 1, :]
        moe = moe + combine[:, e:e + 1] * y
    a = _lrelu(jnp.dot(moe, W0_ref[...], preferred_element_type=jnp.float32)
               + b0_ref[...])
    g = _lrelu(jnp.dot(a, W1d_ref[...], preferred_element_type=jnp.float32)
               + b1d_ref[...])
    s = jnp.sum(g * WoT_ref[...], axis=1, keepdims=True) + bo_ref[...]
    out_ref[...] = 1.0 / (1.0 + jnp.exp(-s))
    frac = (total0 + total1) / (B * K)
    p_mean = jnp.sum(ps_ref[...], axis=(0, 1)) / B
    aux_ref[...] = (E * jnp.sum(frac * p_mean)).reshape(1, 1)


def _moe_head(x, tv, ti, h0, h1, ps, W1, b1, W2, b2, W0, b0, W1d, b1d, WoT, bo):
    return pl.pallas_call(
        _moe_head_body,
        grid=(NB,),
        in_specs=[
            pl.BlockSpec((TB, WAD), lambda i: (i, 0)),
            pl.BlockSpec((TB, K), lambda i: (i, 0)),
            pl.BlockSpec((TB, K), lambda i: (i, 0)),
            pl.BlockSpec((NB, 1, E), lambda i: (0, 0, 0)),
            pl.BlockSpec((NB, 1, E), lambda i: (0, 0, 0)),
            pl.BlockSpec((NB, 1, E), lambda i: (0, 0, 0)),
            pl.BlockSpec((E, WAD, WAD), lambda i: (0, 0, 0)),
            pl.BlockSpec((E, WAD), lambda i: (0, 0)),
            pl.BlockSpec((E, WAD, WAD), lambda i: (0, 0, 0)),
            pl.BlockSpec((E, WAD), lambda i: (0, 0)),
            pl.BlockSpec((WAD, 3 * WAD), lambda i: (0, 0)),
            pl.BlockSpec((1, 3 * WAD), lambda i: (0, 0)),
            pl.BlockSpec((3 * WAD, WAD), lambda i: (0, 0)),
            pl.BlockSpec((1, WAD), lambda i: (0, 0)),
            pl.BlockSpec((1, WAD), lambda i: (0, 0)),
            pl.BlockSpec((1, 1), lambda i: (0, 0)),
        ],
        out_specs=[
            pl.BlockSpec((TB, 1), lambda i: (i, 0)),
            pl.BlockSpec((1, 1), lambda i: (0, 0)),
        ],
        out_shape=[
            jax.ShapeDtypeStruct((B, 1), jnp.float32),
            jax.ShapeDtypeStruct((1, 1), jnp.float32),
        ],
    )(x, tv, ti, h0, h1, ps, W1, b1, W2, b2, W0, b0, W1d, b1d, WoT, bo)


def kernel(deep_in, wide_in, tables, Wd, bd, Ww, bw, Wr, W1, b1, W2, b2,
           W0, b0, W1d, b1d, Wo, bo):
    tabT = tables.swapaxes(1, 2).reshape(FD, VOCAB)
    tab_flat = _repack(tabT).reshape(NG * VP * 8, DE)
    f = jnp.arange(ND, dtype=jnp.int32)
    offs = ((f // 8) * VP * 8 + f % 8)[:, None]
    idx26 = deep_in * 8 + offs
    idx32 = jnp.concatenate([idx26, jnp.zeros((PF - ND, B), jnp.int32)], axis=0)
    idx = idx32.reshape(NG, 8, B).transpose(0, 2, 1).reshape(NWORK * NCHUNK, RPC)
    deep_rows = _sc_gather(tab_flat, idx)
    deep4 = deep_rows.reshape(NG, B, 8 * DE)
    Wdp = jnp.pad(Wd.reshape(ND, DE, HALF),
                  ((0, PF - ND), (0, 0), (0, 0))).reshape(PF * DE, HALF)
    x, tv, ti, h0, h1, ps = _ctx_router(
        deep4, wide_in.T, Wdp, bd.reshape(1, HALF), Ww, bw.reshape(1, HALF), Wr)
    out, aux = _moe_head(
        x, tv, ti, h0, h1, ps, W1, b1, W2, b2, W0, b0.reshape(1, 3 * WAD),
        W1d, b1d.reshape(1, WAD), Wo.reshape(1, WAD), bo.reshape(1, 1))
    return out, aux.reshape(())


# speedup vs baseline: 2.2469x; 2.2469x over previous
import functools

import jax
import jax.numpy as jnp
from jax import lax
from jax.experimental import pallas as pl
from jax.experimental.pallas import tpu as pltpu
from jax.experimental.pallas import tpu_sc as plsc

B = 16384
ND = 26
VOCAB = 100000
DE = 16
NWIDE = 26
WAD = 128
HALF = WAD // 2
E = 8
K = 2
CAP = 4096

NWORK = 32
NCHUNK = 4
PF = 32
TOTAL_ROWS = B * PF
RPC = TOTAL_ROWS // (NWORK * NCHUNK)

FD = ND * DE
NG = 4
CB = 12544
NCB = 8
VP = NCB * CB


def _repack_body(t_ref, p_ref):
    p_ref[...] = t_ref[...].T


def _repack(tabT):
    return pl.pallas_call(
        _repack_body,
        grid=(NG, NCB),
        in_specs=[pl.BlockSpec((128, CB), lambda g, c: (g, c))],
        out_specs=pl.BlockSpec((CB, 128), lambda g, c: (g * NCB + c, 0)),
        out_shape=jax.ShapeDtypeStruct((NG * VP, 128), jnp.float32),
    )(tabT)

@functools.cache
def _make_sc_gather():
    mesh = plsc.VectorSubcoreMesh(
        core_axis_name="c", subcore_axis_name="s", num_cores=2, num_subcores=16
    )

    @functools.partial(
        pl.kernel,
        out_type=jax.ShapeDtypeStruct((NWORK * NCHUNK, RPC, DE), jnp.float32),
        mesh=mesh,
        scratch_types=[
            pltpu.VMEM((RPC,), jnp.int32),
            pltpu.VMEM((RPC, DE), jnp.float32),
            pltpu.SemaphoreType.DMA,
        ],
        compiler_params=pltpu.CompilerParams(use_tc_tiling_on_sc=False),
    )
    def sc_gather(tab_hbm, idx_hbm, out_hbm, idx_v, rows_v, sem):
        w = lax.axis_index("s") * 2 + lax.axis_index("c")
        for c in range(NCHUNK):
            k = w * NCHUNK + c
            pltpu.sync_copy(idx_hbm.at[k], idx_v)
            pltpu.async_copy(tab_hbm.at[idx_v], rows_v, sem).wait()
            pltpu.sync_copy(rows_v, out_hbm.at[k])

    return sc_gather


def _sc_gather(tab_flat, idx):
    return _make_sc_gather()(tab_flat, idx)


TB = 512
NB = B // TB


def _lrelu(v):
    return jnp.where(v >= 0, v, 0.2 * v)


def _ctx_router_body(deep_ref, wide_ref, Wd_ref, bd_ref, Ww_ref, bw_ref, Wr_ref,
                     x_ref, tv_ref, ti_ref, h0_ref, h1_ref, ps_ref):
    dc = jnp.concatenate([deep_ref[g] for g in range(NG)], axis=1)
    d = jnp.dot(dc, Wd_ref[...], preferred_element_type=jnp.float32)
    d = _lrelu(d + bd_ref[...])
    w = jnp.dot(wide_ref[...], Ww_ref[...], preferred_element_type=jnp.float32)
    w = _lrelu(w + bw_ref[...])
    x = jnp.concatenate([d, w], axis=1)
    x_ref[...] = x
    lg = jnp.dot(x, Wr_ref[...], preferred_element_type=jnp.float32)
    m = jnp.max(lg, axis=1, keepdims=True)
    ex = jnp.exp(lg - m)
    p = ex / jnp.sum(ex, axis=1, keepdims=True)
    ii = lax.broadcasted_iota(jnp.int32, p.shape, 1)
    v1 = jnp.max(p, axis=1, keepdims=True)
    i1 = jnp.min(jnp.where(p == v1, ii, E), axis=1, keepdims=True)
    p2 = jnp.where(ii == i1, -1.0, p)
    v2 = jnp.max(p2, axis=1, keepdims=True)
    i2 = jnp.min(jnp.where(p2 == v2, ii, E), axis=1, keepdims=True)
    tv_ref[...] = jnp.concatenate([v1, v2], axis=1)
    ti_ref[...] = jnp.concatenate([i1, i2], axis=1)
    oh0 = (ii == i1).astype(jnp.float32)
    oh1 = (ii == i2).astype(jnp.float32)
    h0_ref[...] = jnp.sum(oh0, axis=0).reshape(1, 1, E)
    h1_ref[...] = jnp.sum(oh1, axis=0).reshape(1, 1, E)
    ps_ref[...] = jnp.sum(p, axis=0).reshape(1, 1, E)


def _ctx_router(deep_cat, wide_T, Wd, bd, Ww, bw, Wr):
    return pl.pallas_call(
        _ctx_router_body,
        grid=(NB,),
        in_specs=[
            pl.BlockSpec((NG, TB, 8 * DE), lambda i: (0, i, 0)),
            pl.BlockSpec((TB, NWIDE), lambda i: (i, 0)),
            pl.BlockSpec((PF * DE, HALF), lambda i: (0, 0)),
            pl.BlockSpec((1, HALF), lambda i: (0, 0)),
            pl.BlockSpec((NWIDE, HALF), lambda i: (0, 0)),
            pl.BlockSpec((1, HALF), lambda i: (0, 0)),
            pl.BlockSpec((WAD, E), lambda i: (0, 0)),
        ],
        out_specs=[
            pl.BlockSpec((TB, WAD), lambda i: (i, 0)),
            pl.BlockSpec((TB, K), lambda i: (i, 0)),
            pl.BlockSpec((TB, K), lambda i: (i, 0)),
            pl.BlockSpec((1, 1, E), lambda i: (i, 0, 0)),
            pl.BlockSpec((1, 1, E), lambda i: (i, 0, 0)),
            pl.BlockSpec((1, 1, E), lambda i: (i, 0, 0)),
        ],
        out_shape=[
            jax.ShapeDtypeStruct((B, WAD), jnp.float32),
            jax.ShapeDtypeStruct((B, K), jnp.float32),
            jax.ShapeDtypeStruct((B, K), jnp.int32),
            jax.ShapeDtypeStruct((NB, 1, E), jnp.float32),
            jax.ShapeDtypeStruct((NB, 1, E), jnp.float32),
            jax.ShapeDtypeStruct((NB, 1, E), jnp.float32),
        ],
    )(deep_cat, wide_T, Wd, bd, Ww, bw, Wr)


def _moe_head_body(x_ref, tv_ref, ti_ref, h0_ref, h1_ref, ps_ref,
                   W1_ref, b1_ref, W2_ref, b2_ref, W0_ref, b0_ref,
                   W1d_ref, b1d_ref, WoT_ref, bo_ref, out_ref, aux_ref):
    i = pl.program_id(0)
    x = x_ref[...]
    tv = tv_ref[...]
    ti = ti_ref[...]
    ii = lax.broadcasted_iota(jnp.int32, (TB, E), 1)
    oh0 = (ii == ti[:, 0:1]).astype(jnp.float32)
    oh1 = (ii == ti[:, 1:2]).astype(jnp.float32)
    h0 = h0_ref[...]
    h1 = h1_ref[...]
    bmask = (lax.broadcasted_iota(jnp.int32, (NB, 1, E), 0) < i).astype(jnp.float32)
    prefix0 = jnp.sum(h0 * bmask, axis=(0, 1))
    prefix1 = jnp.sum(h1 * bmask, axis=(0, 1))
    total0 = jnp.sum(h0, axis=(0, 1))
    total1 = jnp.sum(h1, axis=(0, 1))
    r = lax.broadcasted_iota(jnp.int32, (TB, TB), 0)
    cc = lax.broadcasted_iota(jnp.int32, (TB, TB), 1)
    L = (cc < r).astype(jnp.float32)
    prior0 = lax.dot(L, oh0, precision=lax.Precision.HIGHEST,
                     preferred_element_type=jnp.float32)
    prior1 = lax.dot(L, oh1, precision=lax.Precision.HIGHEST,
                     preferred_element_type=jnp.float32)
    keep0 = oh0 * (prefix0[None, :] + prior0 < CAP).astype(jnp.float32)
    keep1 = oh1 * (total0[None, :] + prefix1[None, :] + prior1 < CAP).astype(jnp.float32)
    combine = tv[:, 0:1] * keep0 + tv[:, 1:2] * keep1
    moe = jnp.zeros((TB, WAD), jnp.float32)
    for e in range(E):
        h = jnp.dot(x, W1_ref[e], preferred_element_type=jnp.float32)
        h = jnp.maximum(h + b1_ref[e:e + 1, :], 0.0)
        y = jnp.dot(h, W2_ref[e], preferred_element_type=jnp.float32)
        y = y + b2_ref[e:e + 1, :]
        moe = moe + combine[:, e:e + 1] * y
    a = _lrelu(jnp.dot(moe, W0_ref[...], preferred_element_type=jnp.float32)
               + b0_ref[...])
    g = _lrelu(jnp.dot(a, W1d_ref[...], preferred_element_type=jnp.float32)
               + b1d_ref[...])
    s = jnp.sum(g * WoT_ref[...], axis=1, keepdims=True) + bo_ref[...]
    out_ref[...] = 1.0 / (1.0 + jnp.exp(-s))
    frac = (total0 + total1) / (B * K)
    p_mean = jnp.sum(ps_ref[...], axis=(0, 1)) / B
    aux_ref[...] = (E * jnp.sum(frac * p_mean)).reshape(1, 1)


def _moe_head(x, tv, ti, h0, h1, ps, W1, b1, W2, b2, W0, b0, W1d, b1d, WoT, bo):
    return pl.pallas_call(
        _moe_head_body,
        grid=(NB,),
        in_specs=[
            pl.BlockSpec((TB, WAD), lambda i: (i, 0)),
            pl.BlockSpec((TB, K), lambda i: (i, 0)),
            pl.BlockSpec((TB, K), lambda i: (i, 0)),
            pl.BlockSpec((NB, 1, E), lambda i: (0, 0, 0)),
            pl.BlockSpec((NB, 1, E), lambda i: (0, 0, 0)),
            pl.BlockSpec((NB, 1, E), lambda i: (0, 0, 0)),
            pl.BlockSpec((E, WAD, WAD), lambda i: (0, 0, 0)),
            pl.BlockSpec((E, WAD), lambda i: (0, 0)),
            pl.BlockSpec((E, WAD, WAD), lambda i: (0, 0, 0)),
            pl.BlockSpec((E, WAD), lambda i: (0, 0)),
            pl.BlockSpec((WAD, 3 * WAD), lambda i: (0, 0)),
            pl.BlockSpec((1, 3 * WAD), lambda i: (0, 0)),
            pl.BlockSpec((3 * WAD, WAD), lambda i: (0, 0)),
            pl.BlockSpec((1, WAD), lambda i: (0, 0)),
            pl.BlockSpec((1, WAD), lambda i: (0, 0)),
            pl.BlockSpec((1, 1), lambda i: (0, 0)),
        ],
        out_specs=[
            pl.BlockSpec((TB, 1), lambda i: (i, 0)),
            pl.BlockSpec((1, 1), lambda i: (0, 0)),
        ],
        out_shape=[
            jax.ShapeDtypeStruct((B, 1), jnp.float32),
            jax.ShapeDtypeStruct((1, 1), jnp.float32),
        ],
    )(x, tv, ti, h0, h1, ps, W1, b1, W2, b2, W0, b0, W1d, b1d, WoT, bo)


def kernel(deep_in, wide_in, tables, Wd, bd, Ww, bw, Wr, W1, b1, W2, b2,
           W0, b0, W1d, b1d, Wo, bo):
    tabT = tables.swapaxes(1, 2).reshape(FD, VOCAB)
    tab_flat = _repack(tabT).reshape(NG * VP * 8, DE)
    f = jnp.arange(ND, dtype=jnp.int32)
    offs = ((f // 8) * VP * 8 + f % 8)[:, None]
    idx26 = deep_in * 8 + offs
    idx32 = jnp.concatenate(
        [idx26, jnp.broadcast_to(idx26[0:1], (PF - ND, B))], axis=0)
    idx = idx32.reshape(NG, 8, B).transpose(0, 2, 1).reshape(NWORK * NCHUNK, RPC)
    deep_rows = _sc_gather(tab_flat, idx)
    deep4 = deep_rows.reshape(NG, B, 8 * DE)
    Wdp = jnp.pad(Wd.reshape(ND, DE, HALF),
                  ((0, PF - ND), (0, 0), (0, 0))).reshape(PF * DE, HALF)
    x, tv, ti, h0, h1, ps = _ctx_router(
        deep4, wide_in.T, Wdp, bd.reshape(1, HALF), Ww, bw.reshape(1, HALF), Wr)
    out, aux = _moe_head(
        x, tv, ti, h0, h1, ps, W1, b1, W2, b2, W0, b0.reshape(1, 3 * WAD),
        W1d, b1d.reshape(1, WAD), Wo.reshape(1, WAD), bo.reshape(1, 1))
    return out, aux.reshape(())


# speedup vs baseline: 2.3993x; 1.0678x over previous
import functools

import jax
import jax.numpy as jnp
from jax import lax
from jax.experimental import pallas as pl
from jax.experimental.pallas import tpu as pltpu
from jax.experimental.pallas import tpu_sc as plsc

B = 16384
ND = 26
VOCAB = 100000
DE = 16
NWIDE = 26
WAD = 128
HALF = WAD // 2
E = 8
K = 2
CAP = 4096

NWORK = 32
NCHUNK = 4
PF = 32
TOTAL_ROWS = B * PF
RPC = TOTAL_ROWS // (NWORK * NCHUNK)

FD = ND * DE
NG = 4
CB = 12544
NCB = 8
VP = NCB * CB


def _repack_body(t_ref, p_ref):
    p_ref[...] = t_ref[...].T


def _repack(tabT):
    return pl.pallas_call(
        _repack_body,
        grid=(NG, NCB),
        in_specs=[pl.BlockSpec((128, CB), lambda g, c: (g, c))],
        out_specs=pl.BlockSpec((CB, 128), lambda g, c: (g * NCB + c, 0)),
        out_shape=jax.ShapeDtypeStruct((NG * VP, 128), jnp.float32),
    )(tabT)

@functools.cache
def _make_sc_gather():
    mesh = plsc.VectorSubcoreMesh(
        core_axis_name="c", subcore_axis_name="s", num_cores=2, num_subcores=16
    )

    @functools.partial(
        pl.kernel,
        out_type=jax.ShapeDtypeStruct((NWORK * NCHUNK, RPC, DE), jnp.float32),
        mesh=mesh,
        scratch_types=[
            pltpu.VMEM((RPC,), jnp.int32),
            pltpu.VMEM((RPC, DE), jnp.float32),
            pltpu.SemaphoreType.DMA,
        ],
        compiler_params=pltpu.CompilerParams(use_tc_tiling_on_sc=False),
    )
    def sc_gather(tab_hbm, idx_hbm, out_hbm, idx_v, rows_v, sem):
        w = lax.axis_index("s") * 2 + lax.axis_index("c")
        for c in range(NCHUNK):
            k = w * NCHUNK + c
            pltpu.sync_copy(idx_hbm.at[k], idx_v)
            pltpu.async_copy(tab_hbm.at[idx_v], rows_v, sem).wait()
            pltpu.sync_copy(rows_v, out_hbm.at[k])

    return sc_gather


def _sc_gather(tab_flat, idx):
    return _make_sc_gather()(tab_flat, idx)


TB = 512
NB = B // TB


def _lrelu(v):
    return jnp.where(v >= 0, v, 0.2 * v)


def _ctx_router_body(deep_ref, wide_ref, Wd_ref, bd_ref, Ww_ref, bw_ref, Wr_ref,
                     x_ref, tv_ref, ti_ref, h0_ref, h1_ref, ps_ref):
    dc = jnp.concatenate([deep_ref[g] for g in range(NG)], axis=1)
    d = jnp.dot(dc, Wd_ref[...], preferred_element_type=jnp.float32)
    d = _lrelu(d + bd_ref[...])
    w = jnp.dot(wide_ref[...], Ww_ref[...], preferred_element_type=jnp.float32)
    w = _lrelu(w + bw_ref[...])
    x = jnp.concatenate([d, w], axis=1)
    x_ref[...] = x
    lg = jnp.dot(x, Wr_ref[...], preferred_element_type=jnp.float32)
    m = jnp.max(lg, axis=1, keepdims=True)
    ex = jnp.exp(lg - m)
    p = ex / jnp.sum(ex, axis=1, keepdims=True)
    ii = lax.broadcasted_iota(jnp.int32, p.shape, 1)
    v1 = jnp.max(p, axis=1, keepdims=True)
    i1 = jnp.min(jnp.where(p == v1, ii, E), axis=1, keepdims=True)
    p2 = jnp.where(ii == i1, -1.0, p)
    v2 = jnp.max(p2, axis=1, keepdims=True)
    i2 = jnp.min(jnp.where(p2 == v2, ii, E), axis=1, keepdims=True)
    tv_ref[...] = jnp.concatenate([v1, v2], axis=1)
    ti_ref[...] = jnp.concatenate([i1, i2], axis=1)
    oh0 = (ii == i1).astype(jnp.float32)
    oh1 = (ii == i2).astype(jnp.float32)
    h0_ref[...] = jnp.sum(oh0, axis=0).reshape(1, 1, E)
    h1_ref[...] = jnp.sum(oh1, axis=0).reshape(1, 1, E)
    ps_ref[...] = jnp.sum(p, axis=0).reshape(1, 1, E)


def _ctx_router(deep_cat, wide_T, Wd, bd, Ww, bw, Wr):
    return pl.pallas_call(
        _ctx_router_body,
        grid=(NB,),
        in_specs=[
            pl.BlockSpec((NG, TB, 8 * DE), lambda i: (0, i, 0)),
            pl.BlockSpec((TB, NWIDE), lambda i: (i, 0)),
            pl.BlockSpec((PF * DE, HALF), lambda i: (0, 0)),
            pl.BlockSpec((1, HALF), lambda i: (0, 0)),
            pl.BlockSpec((NWIDE, HALF), lambda i: (0, 0)),
            pl.BlockSpec((1, HALF), lambda i: (0, 0)),
            pl.BlockSpec((WAD, E), lambda i: (0, 0)),
        ],
        out_specs=[
            pl.BlockSpec((TB, WAD), lambda i: (i, 0)),
            pl.BlockSpec((TB, K), lambda i: (i, 0)),
            pl.BlockSpec((TB, K), lambda i: (i, 0)),
            pl.BlockSpec((1, 1, E), lambda i: (i, 0, 0)),
            pl.BlockSpec((1, 1, E), lambda i: (i, 0, 0)),
            pl.BlockSpec((1, 1, E), lambda i: (i, 0, 0)),
        ],
        out_shape=[
            jax.ShapeDtypeStruct((B, WAD), jnp.float32),
            jax.ShapeDtypeStruct((B, K), jnp.float32),
            jax.ShapeDtypeStruct((B, K), jnp.int32),
            jax.ShapeDtypeStruct((NB, 1, E), jnp.float32),
            jax.ShapeDtypeStruct((NB, 1, E), jnp.float32),
            jax.ShapeDtypeStruct((NB, 1, E), jnp.float32),
        ],
    )(deep_cat, wide_T, Wd, bd, Ww, bw, Wr)


def _moe_head_body(x_ref, tv_ref, ti_ref, h0_ref, h1_ref, ps_ref,
                   W1_ref, b1_ref, W2_ref, b2_ref, W0_ref, b0_ref,
                   W1d_ref, b1d_ref, WoT_ref, bo_ref, out_ref, aux_ref):
    i = pl.program_id(0)
    x = x_ref[...]
    tv = tv_ref[...]
    ti = ti_ref[...]
    ii = lax.broadcasted_iota(jnp.int32, (TB, E), 1)
    oh0 = (ii == ti[:, 0:1]).astype(jnp.float32)
    oh1 = (ii == ti[:, 1:2]).astype(jnp.float32)
    h0 = h0_ref[...]
    h1 = h1_ref[...]
    bmask = (lax.broadcasted_iota(jnp.int32, (NB, 1, E), 0) < i).astype(jnp.float32)
    prefix0 = jnp.sum(h0 * bmask, axis=(0, 1))
    prefix1 = jnp.sum(h1 * bmask, axis=(0, 1))
    total0 = jnp.sum(h0, axis=(0, 1))
    total1 = jnp.sum(h1, axis=(0, 1))
    r = lax.broadcasted_iota(jnp.int32, (TB, TB), 0)
    cc = lax.broadcasted_iota(jnp.int32, (TB, TB), 1)
    L = (cc < r).astype(jnp.float32)
    prior = jnp.dot(L, jnp.concatenate([oh0, oh1], axis=1),
                    preferred_element_type=jnp.float32)
    prior0 = prior[:, :E]
    prior1 = prior[:, E:]
    keep0 = oh0 * (prefix0[None, :] + prior0 < CAP).astype(jnp.float32)
    keep1 = oh1 * (total0[None, :] + prefix1[None, :] + prior1 < CAP).astype(jnp.float32)
    combine = tv[:, 0:1] * keep0 + tv[:, 1:2] * keep1
    h = jnp.dot(x, W1_ref[...], preferred_element_type=jnp.float32)
    h = jnp.maximum(h + b1_ref[...], 0.0)
    hs = h * jnp.repeat(combine, WAD, axis=1)
    moe = jnp.dot(hs, W2_ref[...], preferred_element_type=jnp.float32)
    moe = moe + jnp.dot(combine, b2_ref[...], preferred_element_type=jnp.float32)
    a = _lrelu(jnp.dot(moe, W0_ref[...], preferred_element_type=jnp.float32)
               + b0_ref[...])
    g = _lrelu(jnp.dot(a, W1d_ref[...], preferred_element_type=jnp.float32)
               + b1d_ref[...])
    s = jnp.sum(g * WoT_ref[...], axis=1, keepdims=True) + bo_ref[...]
    out_ref[...] = 1.0 / (1.0 + jnp.exp(-s))
    frac = (total0 + total1) / (B * K)
    p_mean = jnp.sum(ps_ref[...], axis=(0, 1)) / B
    aux_ref[...] = (E * jnp.sum(frac * p_mean)).reshape(1, 1)


def _moe_head(x, tv, ti, h0, h1, ps, W1, b1, W2, b2, W0, b0, W1d, b1d, WoT, bo):
    return pl.pallas_call(
        _moe_head_body,
        grid=(NB,),
        in_specs=[
            pl.BlockSpec((TB, WAD), lambda i: (i, 0)),
            pl.BlockSpec((TB, K), lambda i: (i, 0)),
            pl.BlockSpec((TB, K), lambda i: (i, 0)),
            pl.BlockSpec((NB, 1, E), lambda i: (0, 0, 0)),
            pl.BlockSpec((NB, 1, E), lambda i: (0, 0, 0)),
            pl.BlockSpec((NB, 1, E), lambda i: (0, 0, 0)),
            pl.BlockSpec((WAD, E * WAD), lambda i: (0, 0)),
            pl.BlockSpec((1, E * WAD), lambda i: (0, 0)),
            pl.BlockSpec((E * WAD, WAD), lambda i: (0, 0)),
            pl.BlockSpec((E, WAD), lambda i: (0, 0)),
            pl.BlockSpec((WAD, 3 * WAD), lambda i: (0, 0)),
            pl.BlockSpec((1, 3 * WAD), lambda i: (0, 0)),
            pl.BlockSpec((3 * WAD, WAD), lambda i: (0, 0)),
            pl.BlockSpec((1, WAD), lambda i: (0, 0)),
            pl.BlockSpec((1, WAD), lambda i: (0, 0)),
            pl.BlockSpec((1, 1), lambda i: (0, 0)),
        ],
        out_specs=[
            pl.BlockSpec((TB, 1), lambda i: (i, 0)),
            pl.BlockSpec((1, 1), lambda i: (0, 0)),
        ],
        out_shape=[
            jax.ShapeDtypeStruct((B, 1), jnp.float32),
            jax.ShapeDtypeStruct((1, 1), jnp.float32),
        ],
    )(x, tv, ti, h0, h1, ps, W1, b1, W2, b2, W0, b0, W1d, b1d, WoT, bo)


def kernel(deep_in, wide_in, tables, Wd, bd, Ww, bw, Wr, W1, b1, W2, b2,
           W0, b0, W1d, b1d, Wo, bo):
    tabT = tables.swapaxes(1, 2).reshape(FD, VOCAB)
    tab_flat = _repack(tabT).reshape(NG * VP * 8, DE)
    f = jnp.arange(ND, dtype=jnp.int32)
    offs = ((f // 8) * VP * 8 + f % 8)[:, None]
    idx26 = deep_in * 8 + offs
    idx32 = jnp.concatenate(
        [idx26, jnp.broadcast_to(idx26[0:1], (PF - ND, B))], axis=0)
    idx = idx32.reshape(NG, 8, B).transpose(0, 2, 1).reshape(NWORK * NCHUNK, RPC)
    deep_rows = _sc_gather(tab_flat, idx)
    deep4 = deep_rows.reshape(NG, B, 8 * DE)
    Wdp = jnp.pad(Wd.reshape(ND, DE, HALF),
                  ((0, PF - ND), (0, 0), (0, 0))).reshape(PF * DE, HALF)
    x, tv, ti, h0, h1, ps = _ctx_router(
        deep4, wide_in.T, Wdp, bd.reshape(1, HALF), Ww, bw.reshape(1, HALF), Wr)
    W1c = W1.transpose(1, 0, 2).reshape(WAD, E * WAD)
    out, aux = _moe_head(
        x, tv, ti, h0, h1, ps, W1c, b1.reshape(1, E * WAD),
        W2.reshape(E * WAD, WAD), b2, W0, b0.reshape(1, 3 * WAD),
        W1d, b1d.reshape(1, WAD), Wo.reshape(1, WAD), bo.reshape(1, 1))
    return out, aux.reshape(())


# speedup vs baseline: 2.6489x; 1.1040x over previous
import functools

import jax
import jax.numpy as jnp
from jax import lax
from jax.experimental import pallas as pl
from jax.experimental.pallas import tpu as pltpu
from jax.experimental.pallas import tpu_sc as plsc

B = 16384
ND = 26
VOCAB = 100000
DE = 16
NWIDE = 26
WAD = 128
HALF = WAD // 2
E = 8
K = 2
CAP = 4096

NWORK = 32
NCHUNK = 4
PF = 32
TOTAL_ROWS = B * PF
RPC = TOTAL_ROWS // (NWORK * NCHUNK)

FD = ND * DE
NG = 4
CB = 12544
NCB = 8
VP = NCB * CB


def _repack_body(t_ref, p_ref):
    p_ref[...] = t_ref[...].T


def _repack(tabT):
    return pl.pallas_call(
        _repack_body,
        grid=(NG, NCB),
        in_specs=[pl.BlockSpec((128, CB), lambda g, c: (g, c))],
        out_specs=pl.BlockSpec((CB, 128), lambda g, c: (g * NCB + c, 0)),
        out_shape=jax.ShapeDtypeStruct((NG * VP, 128), jnp.float32),
    )(tabT)

TPC = RPC // 8


@functools.cache
def _make_sc_gather():
    mesh = plsc.VectorSubcoreMesh(
        core_axis_name="c", subcore_axis_name="s", num_cores=2, num_subcores=16
    )

    @functools.partial(
        pl.kernel,
        out_type=jax.ShapeDtypeStruct((NWORK * NCHUNK, RPC, DE), jnp.float32),
        mesh=mesh,
        scratch_types=[
            pltpu.VMEM((8, TPC), jnp.int32),
            pltpu.VMEM((RPC,), jnp.int32),
            pltpu.VMEM((RPC, DE), jnp.float32),
            pltpu.SemaphoreType.DMA,
        ],
        compiler_params=pltpu.CompilerParams(use_tc_tiling_on_sc=False, needs_layout_passes=False),
    )
    def sc_gather(tab_hbm, deep_hbm, out_hbm, dv, idx_v, rows_v, sem):
        w = lax.axis_index("s") * 2 + lax.axis_index("c")
        lane = lax.broadcasted_iota(jnp.int32, (16,), 0)
        vfo = lane % 8
        vht = lane // 8
        for c in range(NCHUNK):
            k = w * NCHUNK + c
            g = k // 32
            b0 = (k % 32) * TPC
            pltpu.sync_copy(deep_hbm.at[pl.ds(g * 8, 8), pl.ds(b0, TPC)], dv)

            def body(i, vtok):
                vals = plsc.load_gather(dv, [vfo, vtok])
                idx_v[pl.ds(i * 16, 16)] = vals * 8 + vfo
                return vtok + 2

            lax.fori_loop(0, RPC // 16, body, vht)
            pltpu.async_copy(tab_hbm.at[idx_v], rows_v, sem).wait()
            pltpu.sync_copy(rows_v, out_hbm.at[k])

    return sc_gather


def _sc_gather(tab_flat, deep_pad):
    return _make_sc_gather()(tab_flat, deep_pad)


TB = 512
NB = B // TB


def _lrelu(v):
    return jnp.where(v >= 0, v, 0.2 * v)


def _ctx_router_body(deep_ref, wide_ref, Wd_ref, bd_ref, Ww_ref, bw_ref, Wr_ref,
                     x_ref, tv_ref, ti_ref, h0_ref, h1_ref, ps_ref):
    dc = jnp.concatenate([deep_ref[g] for g in range(NG)], axis=1)
    d = jnp.dot(dc, Wd_ref[...], preferred_element_type=jnp.float32)
    d = _lrelu(d + bd_ref[...])
    w = jnp.dot(wide_ref[...], Ww_ref[...], preferred_element_type=jnp.float32)
    w = _lrelu(w + bw_ref[...])
    x = jnp.concatenate([d, w], axis=1)
    x_ref[...] = x
    lg = jnp.dot(x, Wr_ref[...], preferred_element_type=jnp.float32)
    m = jnp.max(lg, axis=1, keepdims=True)
    ex = jnp.exp(lg - m)
    p = ex / jnp.sum(ex, axis=1, keepdims=True)
    ii = lax.broadcasted_iota(jnp.int32, p.shape, 1)
    v1 = jnp.max(p, axis=1, keepdims=True)
    i1 = jnp.min(jnp.where(p == v1, ii, E), axis=1, keepdims=True)
    p2 = jnp.where(ii == i1, -1.0, p)
    v2 = jnp.max(p2, axis=1, keepdims=True)
    i2 = jnp.min(jnp.where(p2 == v2, ii, E), axis=1, keepdims=True)
    tv_ref[...] = jnp.concatenate([v1, v2], axis=1)
    ti_ref[...] = jnp.concatenate([i1, i2], axis=1)
    oh0 = (ii == i1).astype(jnp.float32)
    oh1 = (ii == i2).astype(jnp.float32)
    h0_ref[...] = jnp.sum(oh0, axis=0).reshape(1, 1, E)
    h1_ref[...] = jnp.sum(oh1, axis=0).reshape(1, 1, E)
    ps_ref[...] = jnp.sum(p, axis=0).reshape(1, 1, E)


def _ctx_router(deep_cat, wide_T, Wd, bd, Ww, bw, Wr):
    return pl.pallas_call(
        _ctx_router_body,
        grid=(NB,),
        in_specs=[
            pl.BlockSpec((NG, TB, 8 * DE), lambda i: (0, i, 0)),
            pl.BlockSpec((TB, NWIDE), lambda i: (i, 0)),
            pl.BlockSpec((PF * DE, HALF), lambda i: (0, 0)),
            pl.BlockSpec((1, HALF), lambda i: (0, 0)),
            pl.BlockSpec((NWIDE, HALF), lambda i: (0, 0)),
            pl.BlockSpec((1, HALF), lambda i: (0, 0)),
            pl.BlockSpec((WAD, E), lambda i: (0, 0)),
        ],
        out_specs=[
            pl.BlockSpec((TB, WAD), lambda i: (i, 0)),
            pl.BlockSpec((TB, K), lambda i: (i, 0)),
            pl.BlockSpec((TB, K), lambda i: (i, 0)),
            pl.BlockSpec((1, 1, E), lambda i: (i, 0, 0)),
            pl.BlockSpec((1, 1, E), lambda i: (i, 0, 0)),
            pl.BlockSpec((1, 1, E), lambda i: (i, 0, 0)),
        ],
        out_shape=[
            jax.ShapeDtypeStruct((B, WAD), jnp.float32),
            jax.ShapeDtypeStruct((B, K), jnp.float32),
            jax.ShapeDtypeStruct((B, K), jnp.int32),
            jax.ShapeDtypeStruct((NB, 1, E), jnp.float32),
            jax.ShapeDtypeStruct((NB, 1, E), jnp.float32),
            jax.ShapeDtypeStruct((NB, 1, E), jnp.float32),
        ],
    )(deep_cat, wide_T, Wd, bd, Ww, bw, Wr)


def _moe_head_body(x_ref, tv_ref, ti_ref, h0_ref, h1_ref, ps_ref,
                   W1_ref, b1_ref, W2_ref, b2_ref, W0_ref, b0_ref,
                   W1d_ref, b1d_ref, WoT_ref, bo_ref, out_ref, aux_ref):
    i = pl.program_id(0)
    x = x_ref[...]
    tv = tv_ref[...]
    ti = ti_ref[...]
    ii = lax.broadcasted_iota(jnp.int32, (TB, E), 1)
    oh0 = (ii == ti[:, 0:1]).astype(jnp.float32)
    oh1 = (ii == ti[:, 1:2]).astype(jnp.float32)
    h0 = h0_ref[...]
    h1 = h1_ref[...]
    bmask = (lax.broadcasted_iota(jnp.int32, (NB, 1, E), 0) < i).astype(jnp.float32)
    prefix0 = jnp.sum(h0 * bmask, axis=(0, 1))
    prefix1 = jnp.sum(h1 * bmask, axis=(0, 1))
    total0 = jnp.sum(h0, axis=(0, 1))
    total1 = jnp.sum(h1, axis=(0, 1))
    r = lax.broadcasted_iota(jnp.int32, (TB, TB), 0)
    cc = lax.broadcasted_iota(jnp.int32, (TB, TB), 1)
    L = (cc < r).astype(jnp.float32)
    prior = jnp.dot(L, jnp.concatenate([oh0, oh1], axis=1),
                    preferred_element_type=jnp.float32)
    prior0 = prior[:, :E]
    prior1 = prior[:, E:]
    keep0 = oh0 * (prefix0[None, :] + prior0 < CAP).astype(jnp.float32)
    keep1 = oh1 * (total0[None, :] + prefix1[None, :] + prior1 < CAP).astype(jnp.float32)
    combine = tv[:, 0:1] * keep0 + tv[:, 1:2] * keep1
    h = jnp.dot(x, W1_ref[...], preferred_element_type=jnp.float32)
    h = jnp.maximum(h + b1_ref[...], 0.0)
    hs = h * jnp.repeat(combine, WAD, axis=1)
    moe = jnp.dot(hs, W2_ref[...], preferred_element_type=jnp.float32)
    moe = moe + jnp.dot(combine, b2_ref[...], preferred_element_type=jnp.float32)
    a = _lrelu(jnp.dot(moe, W0_ref[...], preferred_element_type=jnp.float32)
               + b0_ref[...])
    g = _lrelu(jnp.dot(a, W1d_ref[...], preferred_element_type=jnp.float32)
               + b1d_ref[...])
    s = jnp.sum(g * WoT_ref[...], axis=1, keepdims=True) + bo_ref[...]
    out_ref[...] = 1.0 / (1.0 + jnp.exp(-s))
    frac = (total0 + total1) / (B * K)
    p_mean = jnp.sum(ps_ref[...], axis=(0, 1)) / B
    aux_ref[...] = (E * jnp.sum(frac * p_mean)).reshape(1, 1)


def _moe_head(x, tv, ti, h0, h1, ps, W1, b1, W2, b2, W0, b0, W1d, b1d, WoT, bo):
    return pl.pallas_call(
        _moe_head_body,
        grid=(NB,),
        in_specs=[
            pl.BlockSpec((TB, WAD), lambda i: (i, 0)),
            pl.BlockSpec((TB, K), lambda i: (i, 0)),
            pl.BlockSpec((TB, K), lambda i: (i, 0)),
            pl.BlockSpec((NB, 1, E), lambda i: (0, 0, 0)),
            pl.BlockSpec((NB, 1, E), lambda i: (0, 0, 0)),
            pl.BlockSpec((NB, 1, E), lambda i: (0, 0, 0)),
            pl.BlockSpec((WAD, E * WAD), lambda i: (0, 0)),
            pl.BlockSpec((1, E * WAD), lambda i: (0, 0)),
            pl.BlockSpec((E * WAD, WAD), lambda i: (0, 0)),
            pl.BlockSpec((E, WAD), lambda i: (0, 0)),
            pl.BlockSpec((WAD, 3 * WAD), lambda i: (0, 0)),
            pl.BlockSpec((1, 3 * WAD), lambda i: (0, 0)),
            pl.BlockSpec((3 * WAD, WAD), lambda i: (0, 0)),
            pl.BlockSpec((1, WAD), lambda i: (0, 0)),
            pl.BlockSpec((1, WAD), lambda i: (0, 0)),
            pl.BlockSpec((1, 1), lambda i: (0, 0)),
        ],
        out_specs=[
            pl.BlockSpec((TB, 1), lambda i: (i, 0)),
            pl.BlockSpec((1, 1), lambda i: (0, 0)),
        ],
        out_shape=[
            jax.ShapeDtypeStruct((B, 1), jnp.float32),
            jax.ShapeDtypeStruct((1, 1), jnp.float32),
        ],
    )(x, tv, ti, h0, h1, ps, W1, b1, W2, b2, W0, b0, W1d, b1d, WoT, bo)


def kernel(deep_in, wide_in, tables, Wd, bd, Ww, bw, Wr, W1, b1, W2, b2,
           W0, b0, W1d, b1d, Wo, bo):
    tabT = tables.swapaxes(1, 2).reshape(FD, VOCAB)
    tab_flat = _repack(tabT).reshape(NG * VP * 8, DE)
    frow = jnp.arange(PF, dtype=jnp.int32)
    goff = jnp.where(frow < ND, (frow // 8) * VP, 0)[:, None]
    deep_pad = jnp.concatenate(
        [deep_in, jnp.broadcast_to(deep_in[0:1], (PF - ND, B))], axis=0) + goff
    deep_rows = _sc_gather(tab_flat, deep_pad)
    deep4 = deep_rows.reshape(NG, B, 8 * DE)
    Wdp = jnp.pad(Wd.reshape(ND, DE, HALF),
                  ((0, PF - ND), (0, 0), (0, 0))).reshape(PF * DE, HALF)
    x, tv, ti, h0, h1, ps = _ctx_router(
        deep4, wide_in.T, Wdp, bd.reshape(1, HALF), Ww, bw.reshape(1, HALF), Wr)
    W1c = W1.transpose(1, 0, 2).reshape(WAD, E * WAD)
    out, aux = _moe_head(
        x, tv, ti, h0, h1, ps, W1c, b1.reshape(1, E * WAD),
        W2.reshape(E * WAD, WAD), b2, W0, b0.reshape(1, 3 * WAD),
        W1d, b1d.reshape(1, WAD), Wo.reshape(1, WAD), bo.reshape(1, 1))
    return out, aux.reshape(())
